# baseline (speedup 1.0000x reference)
import functools

import jax
import jax.numpy as jnp
from jax import lax
from jax.experimental import pallas as pl
from jax.experimental.pallas import tpu as pltpu
from jax.experimental.pallas import tpu_sc as plsc

D_MODEL = 768
FF = 3072
N_EXP = 8
TOPK = 2
T_TOK = 2048
NA = T_TOK * TOPK
TILE = 256
NT = NA // TILE + N_EXP
CH = 256
NEG = -1e30

NW = 32
APW = NA // NW
TPW = T_TOK // NW
SUB = 32


def _route_body(x_ref, wg_ref, probs_ref, pos_ref, etile_ref, valid_ref):
    xf = x_ref[...]
    wg = wg_ref[...]
    scores = lax.dot_general(xf, wg, (((1,), (1,)), ((), ())),
                             preferred_element_type=jnp.float32)
    col = lax.broadcasted_iota(jnp.int32, (T_TOK, N_EXP), 1)
    m0 = jnp.max(scores, axis=1, keepdims=True)
    i0 = jnp.min(jnp.where(scores == m0, col, N_EXP), axis=1,
                 keepdims=True)
    masked = jnp.where(col == i0, NEG, scores)
    m1 = jnp.max(masked, axis=1, keepdims=True)
    i1 = jnp.min(jnp.where(masked == m1, col, N_EXP), axis=1,
                 keepdims=True)
    e1 = jnp.exp(m1 - m0)
    p0 = 1.0 / (1.0 + e1)
    probs_ref[...] = jnp.concatenate([p0, e1 * p0], axis=1)

    one0 = (col == i0).astype(jnp.float32)
    one1 = (col == i1).astype(jnp.float32)
    both = one0 + one1
    r = lax.broadcasted_iota(jnp.int32, (CH, CH), 0)
    c = lax.broadcasted_iota(jnp.int32, (CH, CH), 1)
    tril = (c < r).astype(jnp.float32)
    base = jnp.zeros((1, N_EXP), jnp.float32)
    chunks = []
    for k in range(T_TOK // CH):
        blk = both[k * CH:(k + 1) * CH, :]
        chunks.append(jnp.dot(tril, blk, preferred_element_type=jnp.float32)
                      + base)
        base = base + jnp.sum(blk, axis=0, keepdims=True)
    s_excl = jnp.concatenate(chunks, axis=0)
    counts = base
    pc = jnp.ceil(counts / TILE) * TILE
    er = lax.broadcasted_iota(jnp.int32, (N_EXP, N_EXP), 0)
    ec = lax.broadcasted_iota(jnp.int32, (N_EXP, N_EXP), 1)
    offs = jnp.dot(pc, (er < ec).astype(jnp.float32),
                   preferred_element_type=jnp.float32)
    slot = offs + s_excl
    pos0 = jnp.sum(one0 * slot, axis=1, keepdims=True)
    pos1 = jnp.sum(one1 * slot, axis=1, keepdims=True)
    pos_ref[...] = jnp.concatenate([pos0, pos1], axis=1).astype(jnp.int32)

    ends = offs + pc
    tstart = (lax.broadcasted_iota(jnp.int32, (NT, 1), 0)
              .astype(jnp.float32) * TILE)
    raw = jnp.sum((tstart >= ends).astype(jnp.float32), axis=1, keepdims=True)
    lane = lax.broadcasted_iota(jnp.int32, (1, N_EXP), 1).astype(jnp.float32)
    last_used = jnp.max(jnp.where(counts > 0, lane, -1.0))
    etile_ref[...] = jnp.minimum(raw, last_used).astype(jnp.int32)
    valid_ref[...] = (tstart < jnp.sum(pc)).astype(jnp.int32)


def _route(x_flat, wg):
    return pl.pallas_call(
        _route_body,
        out_shape=[
            jax.ShapeDtypeStruct((T_TOK, TOPK), jnp.float32),
            jax.ShapeDtypeStruct((T_TOK, TOPK), jnp.int32),
            jax.ShapeDtypeStruct((NT, 1), jnp.int32),
            jax.ShapeDtypeStruct((NT, 1), jnp.int32),
        ],
    )(x_flat, wg)


def _ffn_body(et_ref, va_ref, xg_ref, w1_ref, w2_ref, out_ref):
    i = pl.program_id(0)

    @pl.when(va_ref[i, 0] == 1)
    def _():
        h = lax.dot_general(xg_ref[...], w1_ref[0], (((1,), (1,)), ((), ())),
                            preferred_element_type=jnp.float32)
        h = h * jax.nn.sigmoid(h)
        out_ref[...] = lax.dot_general(h, w2_ref[0], (((1,), (1,)), ((), ())),
                                       preferred_element_type=jnp.float32)


def _ffn(etile, valid, xg, w1, w2):
    return pl.pallas_call(
        _ffn_body,
        grid_spec=pltpu.PrefetchScalarGridSpec(
            num_scalar_prefetch=2,
            grid=(NT,),
            in_specs=[
                pl.BlockSpec((TILE, D_MODEL), lambda i, et, va: (i, 0)),
                pl.BlockSpec((1, FF, D_MODEL),
                             lambda i, et, va: (et[i, 0], 0, 0)),
                pl.BlockSpec((1, D_MODEL, FF),
                             lambda i, et, va: (et[i, 0], 0, 0)),
            ],
            out_specs=pl.BlockSpec((TILE, D_MODEL), lambda i, et, va: (i, 0)),
        ),
        out_shape=jax.ShapeDtypeStruct((NT * TILE, D_MODEL), jnp.float32),
    )(etile, valid, xg, w1, w2)


def _wsum_body(r2_ref, p_ref, y_ref):
    p = p_ref[...]
    y_ref[...] = (p[:, 0:1] * r2_ref[:, :D_MODEL]
                  + p[:, 1:2] * r2_ref[:, D_MODEL:])


def _wsum(r2, probs):
    return pl.pallas_call(
        _wsum_body,
        grid=(T_TOK // TILE,),
        in_specs=[
            pl.BlockSpec((TILE, 2 * D_MODEL), lambda i: (i, 0)),
            pl.BlockSpec((TILE, TOPK), lambda i: (i, 0)),
        ],
        out_specs=pl.BlockSpec((TILE, D_MODEL), lambda i: (i, 0)),
        out_shape=jax.ShapeDtypeStruct((T_TOK, D_MODEL), jnp.float32),
    )(r2, probs)


@functools.cache
def _sc_mesh():
    return plsc.VectorSubcoreMesh(core_axis_name="c", subcore_axis_name="s",
                                  num_cores=2)


@functools.cache
def _dispatch_kernel():
    return functools.partial(
        pl.kernel, mesh=_sc_mesh(),
        out_type=jax.ShapeDtypeStruct((NT * TILE, D_MODEL), jnp.float32),
        scratch_types=[
            pltpu.VMEM((APW,), jnp.int32),
            pltpu.VMEM((APW,), jnp.int32),
            pltpu.VMEM((APW, D_MODEL), jnp.float32),
            pltpu.SemaphoreType.DMA,
        ],
    )(_dispatch_body)


def _dispatch_body(x_hbm, pos_hbm, xg_hbm, tok_v, pos_v, rows_v, sem):
    wid = lax.axis_index("s") * 2 + lax.axis_index("c")
    base = wid * APW
    pltpu.sync_copy(pos_hbm.at[pl.ds(base, APW)], pos_v)

    def bld(j, carry):
        idx16 = (jnp.full((16,), base + j * 16, jnp.int32)
                 + lax.iota(jnp.int32, 16)) >> 1
        tok_v[pl.ds(j * 16, 16)] = idx16
        return carry

    lax.fori_loop(0, APW // 16, bld, 0)
    pltpu.async_copy(x_hbm.at[tok_v], rows_v, sem).wait()
    pltpu.async_copy(rows_v, xg_hbm.at[pos_v], sem).wait()


@functools.cache
def _combine_kernel():
    return functools.partial(
        pl.kernel, mesh=_sc_mesh(),
        out_type=jax.ShapeDtypeStruct((T_TOK, D_MODEL), jnp.float32),
        scratch_types=[
            pltpu.VMEM((2 * TPW,), jnp.int32),
            pltpu.VMEM((2 * TPW,), jnp.float32),
            pltpu.VMEM((2 * TPW, D_MODEL), jnp.float32),
            pltpu.VMEM((TPW // 2, D_MODEL), jnp.float32),
            pltpu.SemaphoreType.DMA,
            pltpu.SemaphoreType.DMA,
        ],
    )(_combine_body)


def _combine_body(outs_hbm, posf_hbm, probsf_hbm, y_hbm, idx_v, p_v, r_v, y_v,
                  sem0, sem1):
    wid = lax.axis_index("s") * 2 + lax.axis_index("c")
    tb = wid * TPW
    half = TPW
    pltpu.sync_copy(posf_hbm.at[pl.ds(tb * 2, 2 * TPW)], idx_v)
    pltpu.sync_copy(probsf_hbm.at[pl.ds(tb * 2, 2 * TPW)], p_v)
    cp0 = pltpu.async_copy(outs_hbm.at[idx_v.at[pl.ds(0, half)]],
                           r_v.at[pl.ds(0, half)], sem0)
    cp1 = pltpu.async_copy(outs_hbm.at[idx_v.at[pl.ds(half, half)]],
                           r_v.at[pl.ds(half, half)], sem1)
    for h, cp in ((0, cp0), (1, cp1)):
        cp.wait()
        base = h * half

        def grp(g, c2, base=base):
            pvec = p_v[pl.ds(base + g * 16, 16)]

            def row(i2, c3):
                i = g * 8 + i2
                p0 = pvec.at[jnp.full((16,), 2 * i2, jnp.int32)].get(
                    mode="promise_in_bounds")
                p1 = pvec.at[jnp.full((16,), 2 * i2 + 1, jnp.int32)].get(
                    mode="promise_in_bounds")

                for cc in range(D_MODEL // 16):
                    a = r_v[base + 2 * i, pl.ds(cc * 16, 16)]
                    b = r_v[base + 2 * i + 1, pl.ds(cc * 16, 16)]
                    y_v[i, pl.ds(cc * 16, 16)] = a * p0 + b * p1
                return c3

            lax.fori_loop(0, 8, row, 0)
            return c2

        lax.fori_loop(0, half // 16, grp, 0)
        pltpu.sync_copy(y_v, y_hbm.at[pl.ds(tb + h * (half // 2), half // 2)])


def kernel(x, Wg, W1, W2):
    b, t, c = x.shape
    x_flat = x.reshape(t, c)
    probs, pos, etile, valid = _route(x_flat, Wg)
    xg = _dispatch_kernel()(x_flat, pos.reshape(-1))
    out_s = _ffn(etile, valid, xg, W1, W2)
    y = _combine_kernel()(out_s, pos.reshape(-1), probs.reshape(-1))
    return y.reshape(b, t, c)


# speedup vs baseline: 1.0061x; 1.0061x over previous
import functools

import jax
import jax.numpy as jnp
from jax import lax
from jax.experimental import pallas as pl
from jax.experimental.pallas import tpu as pltpu
from jax.experimental.pallas import tpu_sc as plsc

D_MODEL = 768
FF = 3072
N_EXP = 8
TOPK = 2
T_TOK = 2048
NA = T_TOK * TOPK
TILE = 256
NT = NA // TILE + N_EXP
CH = 256
NEG = -1e30

NW = 32
APW = NA // NW
TPW = T_TOK // NW


def _route_body(x_ref, wg_ref, probs_ref, pos_ref, etile_ref, valid_ref):
    xf = x_ref[...]
    wg = wg_ref[...]
    scores = lax.dot_general(xf, wg, (((1,), (1,)), ((), ())),
                             preferred_element_type=jnp.float32)
    col = lax.broadcasted_iota(jnp.int32, (T_TOK, N_EXP), 1)
    m0 = jnp.max(scores, axis=1, keepdims=True)
    i0 = jnp.min(jnp.where(scores == m0, col, N_EXP), axis=1,
                 keepdims=True)
    masked = jnp.where(col == i0, NEG, scores)
    m1 = jnp.max(masked, axis=1, keepdims=True)
    i1 = jnp.min(jnp.where(masked == m1, col, N_EXP), axis=1,
                 keepdims=True)
    e1 = jnp.exp(m1 - m0)
    p0 = 1.0 / (1.0 + e1)
    probs_ref[...] = jnp.concatenate([p0, e1 * p0], axis=1)

    one0 = (col == i0).astype(jnp.float32)
    one1 = (col == i1).astype(jnp.float32)
    both = one0 + one1
    r = lax.broadcasted_iota(jnp.int32, (CH, CH), 0)
    c = lax.broadcasted_iota(jnp.int32, (CH, CH), 1)
    tril = (c < r).astype(jnp.float32)
    base = jnp.zeros((1, N_EXP), jnp.float32)
    chunks = []
    for k in range(T_TOK // CH):
        blk = both[k * CH:(k + 1) * CH, :]
        chunks.append(jnp.dot(tril, blk, preferred_element_type=jnp.float32)
                      + base)
        base = base + jnp.sum(blk, axis=0, keepdims=True)
    s_excl = jnp.concatenate(chunks, axis=0)
    counts = base
    pc = jnp.ceil(counts / TILE) * TILE
    er = lax.broadcasted_iota(jnp.int32, (N_EXP, N_EXP), 0)
    ec = lax.broadcasted_iota(jnp.int32, (N_EXP, N_EXP), 1)
    offs = jnp.dot(pc, (er < ec).astype(jnp.float32),
                   preferred_element_type=jnp.float32)
    slot = offs + s_excl
    pos0 = jnp.sum(one0 * slot, axis=1, keepdims=True)
    pos1 = jnp.sum(one1 * slot, axis=1, keepdims=True)
    pos_ref[...] = jnp.concatenate([pos0, pos1], axis=1).astype(jnp.int32)

    ends = offs + pc
    tstart = (lax.broadcasted_iota(jnp.int32, (NT, 1), 0)
              .astype(jnp.float32) * TILE)
    raw = jnp.sum((tstart >= ends).astype(jnp.float32), axis=1, keepdims=True)
    lane = lax.broadcasted_iota(jnp.int32, (1, N_EXP), 1).astype(jnp.float32)
    last_used = jnp.max(jnp.where(counts > 0, lane, -1.0))
    etile_ref[...] = jnp.minimum(raw, last_used).astype(jnp.int32)
    valid_ref[...] = (tstart < jnp.sum(pc)).astype(jnp.int32)


def _route(x_flat, wg):
    return pl.pallas_call(
        _route_body,
        out_shape=[
            jax.ShapeDtypeStruct((T_TOK, TOPK), jnp.float32),
            jax.ShapeDtypeStruct((T_TOK, TOPK), jnp.int32),
            jax.ShapeDtypeStruct((NT, 1), jnp.int32),
            jax.ShapeDtypeStruct((NT, 1), jnp.int32),
        ],
    )(x_flat, wg)


def _ffn_body(et_ref, va_ref, xg_ref, w1_ref, w2_ref, out_ref):
    i = pl.program_id(0)

    @pl.when(va_ref[i, 0] == 1)
    def _():
        h = lax.dot_general(xg_ref[...], w1_ref[0], (((1,), (1,)), ((), ())),
                            preferred_element_type=jnp.float32)
        h = h * jax.nn.sigmoid(h)
        out_ref[...] = lax.dot_general(h, w2_ref[0], (((1,), (1,)), ((), ())),
                                       preferred_element_type=jnp.float32)


def _ffn(etile, valid, xg, w1, w2):
    return pl.pallas_call(
        _ffn_body,
        grid_spec=pltpu.PrefetchScalarGridSpec(
            num_scalar_prefetch=2,
            grid=(NT,),
            in_specs=[
                pl.BlockSpec((TILE, D_MODEL), lambda i, et, va: (i, 0)),
                pl.BlockSpec((1, FF, D_MODEL),
                             lambda i, et, va: (et[i, 0], 0, 0)),
                pl.BlockSpec((1, D_MODEL, FF),
                             lambda i, et, va: (et[i, 0], 0, 0)),
            ],
            out_specs=pl.BlockSpec((TILE, D_MODEL), lambda i, et, va: (i, 0)),
        ),
        out_shape=jax.ShapeDtypeStruct((NT * TILE, D_MODEL), jnp.float32),
    )(etile, valid, xg, w1, w2)


@functools.cache
def _sc_mesh():
    return plsc.VectorSubcoreMesh(core_axis_name="c", subcore_axis_name="s",
                                  num_cores=2)


@functools.cache
def _dispatch_kernel():
    return functools.partial(
        pl.kernel, mesh=_sc_mesh(),
        out_type=jax.ShapeDtypeStruct((NT * TILE, D_MODEL), jnp.float32),
        scratch_types=[
            pltpu.VMEM((APW,), jnp.int32),
            pltpu.VMEM((APW,), jnp.int32),
            pltpu.VMEM((APW, D_MODEL), jnp.float32),
            pltpu.SemaphoreType.DMA,
        ],
    )(_dispatch_body)


def _dispatch_body(x_hbm, pos_hbm, xg_hbm, tok_v, pos_v, rows_v, sem):
    wid = lax.axis_index("s") * 2 + lax.axis_index("c")
    base = wid * APW
    pltpu.sync_copy(pos_hbm.at[pl.ds(base, APW)], pos_v)

    def bld(j, carry):
        idx16 = (jnp.full((16,), base + j * 16, jnp.int32)
                 + lax.iota(jnp.int32, 16)) >> 1
        tok_v[pl.ds(j * 16, 16)] = idx16
        return carry

    lax.fori_loop(0, APW // 16, bld, 0)
    pltpu.async_copy(x_hbm.at[tok_v], rows_v, sem).wait()
    pltpu.async_copy(rows_v, xg_hbm.at[pos_v], sem).wait()


@functools.cache
def _combine_kernel():
    return functools.partial(
        pl.kernel, mesh=_sc_mesh(),
        out_type=jax.ShapeDtypeStruct((T_TOK, D_MODEL), jnp.float32),
        scratch_types=[
            pltpu.VMEM((2 * TPW,), jnp.int32),
            pltpu.VMEM((2 * TPW,), jnp.float32),
            pltpu.VMEM((2 * TPW, D_MODEL), jnp.float32),
            pltpu.SemaphoreType.DMA,
            pltpu.SemaphoreType.DMA,
            pltpu.SemaphoreType.DMA,
        ],
    )(_combine_body)


def _combine_body(outs_hbm, posf_hbm, probsf_hbm, y_hbm, idx_v, p_v, r_v,
                  sem0, sem1, semw):
    wid = lax.axis_index("s") * 2 + lax.axis_index("c")
    tb = wid * TPW
    half = TPW
    pltpu.sync_copy(posf_hbm.at[pl.ds(tb * 2, 2 * TPW)], idx_v)
    pltpu.sync_copy(probsf_hbm.at[pl.ds(tb * 2, 2 * TPW)], p_v)
    cp0 = pltpu.async_copy(outs_hbm.at[idx_v.at[pl.ds(0, half)]],
                           r_v.at[pl.ds(0, half)], sem0)
    cp1 = pltpu.async_copy(outs_hbm.at[idx_v.at[pl.ds(half, half)]],
                           r_v.at[pl.ds(half, half)], sem1)
    writes = []
    for h, cp in ((0, cp0), (1, cp1)):
        cp.wait()
        base = h * half

        def grp(g, c2, base=base):
            pvec = p_v[pl.ds(base + g * 16, 16)]

            def row(i2, c3):
                i = g * 8 + i2
                p0 = pvec.at[jnp.full((16,), 2 * i2, jnp.int32)].get(
                    mode="promise_in_bounds")
                p1 = pvec.at[jnp.full((16,), 2 * i2 + 1, jnp.int32)].get(
                    mode="promise_in_bounds")

                for cc in range(D_MODEL // 16):
                    a = r_v[base + 2 * i, pl.ds(cc * 16, 16)]
                    b = r_v[base + 2 * i + 1, pl.ds(cc * 16, 16)]
                    r_v[base + i, pl.ds(cc * 16, 16)] = a * p0 + b * p1
                return c3

            lax.fori_loop(0, 8, row, 0)
            return c2

        lax.fori_loop(0, half // 16, grp, 0)
        writes.append(pltpu.async_copy(
            r_v.at[pl.ds(base, half // 2)],
            y_hbm.at[pl.ds(tb + h * (half // 2), half // 2)], semw))
    for w in writes:
        w.wait()


def kernel(x, Wg, W1, W2):
    b, t, c = x.shape
    x_flat = x.reshape(t, c)
    probs, pos, etile, valid = _route(x_flat, Wg)
    xg = _dispatch_kernel()(x_flat, pos.reshape(-1))
    out_s = _ffn(etile, valid, xg, W1, W2)
    y = _combine_kernel()(out_s, pos.reshape(-1), probs.reshape(-1))
    return y.reshape(b, t, c)
